# Initial kernel scaffold; baseline (speedup 1.0000x reference)
import jax
import jax.numpy as jnp
from jax.experimental import pallas as pl


def kernel(inputs, embeddings):
    raise NotImplementedError("write your pallas kernel here")


# speedup vs baseline: 44.7574x; 44.7574x over previous
import functools

import jax
import jax.numpy as jnp
import numpy as np
from jax import lax
from jax.experimental import pallas as pl
from jax.experimental.pallas import tpu as pltpu
from jax.experimental.pallas import tpu_sc as plsc

B, D, T = 16, 64, 1024
N = 1024
COMMITMENT_COST = 0.25


_I0 = np.int32(0)


_TT = 512


def _tc_body(x_ref, emb_ref, idx_ref, loss_ref):
    x = x_ref[0]
    emb = emb_ref[...]
    b2 = jnp.sum(emb * emb, axis=1)
    m = lax.dot_general(
        emb, x, (((1,), (0,)), ((), ())),
        preferred_element_type=jnp.float32,
        precision=lax.Precision.HIGHEST)
    s = b2[:, None] - 2.0 * m
    smin = jnp.min(s, axis=0)
    jota = lax.broadcasted_iota(jnp.int32, (N, _TT), 0)
    idx = jnp.min(jnp.where(s == smin[None, :], jota, jnp.int32(N)), axis=0)
    idx_ref[0, 0] = idx
    a2 = jnp.sum(x * x, axis=0)
    part = jnp.sum(smin + a2)

    @pl.when((pl.program_id(0) == 0) & (pl.program_id(1) == 0))
    def _():
        loss_ref[...] = jnp.zeros((1, 1), jnp.float32)

    loss_ref[...] += part.reshape(1, 1)


def _tc_argmin(inputs, embeddings, interpret=False):
    return pl.pallas_call(
        _tc_body,
        grid=(B, T // _TT),
        in_specs=[
            pl.BlockSpec((1, D, _TT), lambda i, j: (i, _I0, j)),
            pl.BlockSpec((N, D), lambda i, j: (_I0, _I0)),
        ],
        out_specs=[
            pl.BlockSpec((1, 1, _TT), lambda i, j: (i, _I0, j)),
            pl.BlockSpec((1, 1), lambda i, j: (_I0, _I0)),
        ],
        out_shape=[
            jax.ShapeDtypeStruct((B, 1, T), jnp.int32),
            jax.ShapeDtypeStruct((1, 1), jnp.float32),
        ],
        interpret=interpret,
    )(inputs, embeddings)


_NC, _NS, _L = 2, 16, 16
_NW = _NC * _NS
_D_PER_W = D // _NW


def _sc_body(embt_hbm, idx_hbm, out_hbm, rows_v, idx_v, obuf_v):
    wid = lax.axis_index("s") * jnp.int32(_NC) + lax.axis_index("c")
    d0 = wid * jnp.int32(_D_PER_W)
    pltpu.sync_copy(embt_hbm.at[pl.ds(d0 * jnp.int32(N), _D_PER_W * N)],
                    rows_v)

    def batch_body(b, _):
        pltpu.sync_copy(idx_hbm.at[pl.ds(b * jnp.int32(T), T)], idx_v)

        def tile_body(i, _):
            t0 = pl.multiple_of(i * jnp.int32(_L), _L)
            idx16 = idx_v[pl.ds(t0, _L)]
            for dd in range(_D_PER_W):
                vals = plsc.load_gather(
                    rows_v, [idx16 + jnp.int32(dd * N)])
                obuf_v[pl.ds(t0 + jnp.int32(dd * T), _L)] = vals
            return jnp.int32(0)

        lax.fori_loop(jnp.int32(0), jnp.int32(T // _L), tile_body, jnp.int32(0))
        pltpu.sync_copy(
            obuf_v,
            out_hbm.at[pl.ds((b * jnp.int32(D) + d0) * jnp.int32(T),
                             _D_PER_W * T)])
        return jnp.int32(0)

    lax.fori_loop(jnp.int32(0), jnp.int32(B), batch_body, jnp.int32(0))


@functools.cache
def _sc_gather_fn():
    return pl.kernel(
        _sc_body,
        out_type=jax.ShapeDtypeStruct((B * D * T,), jnp.float32),
        mesh=plsc.VectorSubcoreMesh(core_axis_name="c", subcore_axis_name="s"),
        scratch_types=[
            pltpu.VMEM((_D_PER_W * N,), jnp.float32),
            pltpu.VMEM((T,), jnp.int32),
            pltpu.VMEM((_D_PER_W * T,), jnp.float32),
        ],
        compiler_params=pltpu.CompilerParams(needs_layout_passes=False),
    )


def kernel(inputs, embeddings):
    idx3, loss_sum = _tc_argmin(inputs, embeddings)
    idx = idx3.reshape(B, T)
    quantized = _sc_gather_fn()(
        embeddings.T.reshape(-1), idx.reshape(-1)).reshape(B, D, T)
    mse = loss_sum[0, 0] / jnp.float32(B * D * T)
    loss = mse + COMMITMENT_COST * mse
    encoding_indices = idx.reshape(B * T).astype(jnp.int64)
    return (quantized, loss, mse, mse, encoding_indices)


# speedup vs baseline: 72.4095x; 1.6178x over previous
import functools

import jax
import jax.numpy as jnp
import numpy as np
from jax import lax
from jax.experimental import pallas as pl
from jax.experimental.pallas import tpu as pltpu
from jax.experimental.pallas import tpu_sc as plsc

B, D, T = 16, 64, 1024
N = 1024
COMMITMENT_COST = 0.25


_I0 = np.int32(0)


_TT = 1024


def _tc_body(x_ref, emb_ref, idx_ref, loss_ref):
    x = x_ref[0]
    emb = emb_ref[...]
    b2 = jnp.sum(emb * emb, axis=1)
    m = lax.dot_general(
        emb, x, (((1,), (0,)), ((), ())),
        preferred_element_type=jnp.float32,
        precision=lax.Precision.DEFAULT)
    s = b2[:, None] - 2.0 * m
    smin = jnp.min(s, axis=0)
    jota = lax.broadcasted_iota(jnp.int32, (N, _TT), 0)
    idx = jnp.min(jnp.where(s == smin[None, :], jota, jnp.int32(N)), axis=0)
    idx_ref[0, 0] = idx
    a2 = jnp.sum(x * x, axis=0)
    part = jnp.sum(smin + a2)

    @pl.when((pl.program_id(0) == 0) & (pl.program_id(1) == 0))
    def _():
        loss_ref[...] = jnp.zeros((1, 1), jnp.float32)

    loss_ref[...] += part.reshape(1, 1)


def _tc_argmin(inputs, embeddings, interpret=False):
    return pl.pallas_call(
        _tc_body,
        grid=(B, T // _TT),
        in_specs=[
            pl.BlockSpec((1, D, _TT), lambda i, j: (i, _I0, j)),
            pl.BlockSpec((N, D), lambda i, j: (_I0, _I0)),
        ],
        out_specs=[
            pl.BlockSpec((1, 1, _TT), lambda i, j: (i, _I0, j)),
            pl.BlockSpec((1, 1), lambda i, j: (_I0, _I0)),
        ],
        out_shape=[
            jax.ShapeDtypeStruct((B, 1, T), jnp.int32),
            jax.ShapeDtypeStruct((1, 1), jnp.float32),
        ],
        interpret=interpret,
    )(inputs, embeddings)


_NC, _NS, _L = 2, 16, 16
_NW = _NC * _NS
_D_PER_W = D // _NW


def _sc_body(embt_hbm, idx_hbm, out_hbm, rows_v, idx_v, obuf_v):
    wid = lax.axis_index("s") * jnp.int32(_NC) + lax.axis_index("c")
    d0 = wid * jnp.int32(_D_PER_W)
    pltpu.sync_copy(embt_hbm.at[pl.ds(d0 * jnp.int32(N), _D_PER_W * N)],
                    rows_v)

    def batch_body(b, _):
        pltpu.sync_copy(idx_hbm.at[pl.ds(b * jnp.int32(T), T)], idx_v)

        def tile_body(i, _):
            t0 = pl.multiple_of(i * jnp.int32(_L), _L)
            idx16 = idx_v[pl.ds(t0, _L)]
            for dd in range(_D_PER_W):
                vals = plsc.load_gather(
                    rows_v, [idx16 + jnp.int32(dd * N)])
                obuf_v[pl.ds(t0 + jnp.int32(dd * T), _L)] = vals
            return jnp.int32(0)

        lax.fori_loop(jnp.int32(0), jnp.int32(T // _L), tile_body, jnp.int32(0))
        pltpu.sync_copy(
            obuf_v,
            out_hbm.at[pl.ds((b * jnp.int32(D) + d0) * jnp.int32(T),
                             _D_PER_W * T)])
        return jnp.int32(0)

    lax.fori_loop(jnp.int32(0), jnp.int32(B), batch_body, jnp.int32(0))


@functools.cache
def _sc_gather_fn():
    return pl.kernel(
        _sc_body,
        out_type=jax.ShapeDtypeStruct((B * D * T,), jnp.float32),
        mesh=plsc.VectorSubcoreMesh(core_axis_name="c", subcore_axis_name="s"),
        scratch_types=[
            pltpu.VMEM((_D_PER_W * N,), jnp.float32),
            pltpu.VMEM((T,), jnp.int32),
            pltpu.VMEM((_D_PER_W * T,), jnp.float32),
        ],
        compiler_params=pltpu.CompilerParams(needs_layout_passes=False),
    )


def kernel(inputs, embeddings):
    idx3, loss_sum = _tc_argmin(inputs, embeddings)
    idx = idx3.reshape(B, T)
    quantized = _sc_gather_fn()(
        embeddings.T.reshape(-1), idx.reshape(-1)).reshape(B, D, T)
    mse = loss_sum[0, 0] / jnp.float32(B * D * T)
    loss = mse + COMMITMENT_COST * mse
    encoding_indices = idx.reshape(B * T).astype(jnp.int64)
    return (quantized, loss, mse, mse, encoding_indices)
